# Initial kernel scaffold; baseline (speedup 1.0000x reference)
import jax
import jax.numpy as jnp
from jax import lax
from jax.experimental import pallas as pl
from jax.experimental.pallas import tpu as pltpu, tpu_sc as plsc

D = 64
S = 200
B = 4096
NC = 2
NS = 16
NW = NC * NS
ROWS_PER_W = B // NW
CH = S // 2


def _body(ids_hbm, pos_ids_hbm, tok_hbm, pos_hbm, out_hbm, idx_v, pidx_v, buf):
    wid = lax.axis_index("s") * NC + lax.axis_index("c")
    row0 = wid * ROWS_PER_W
    pltpu.sync_copy(pos_ids_hbm, pidx_v)

    def row(r, carry):
        gr = row0 + r
        pltpu.sync_copy(ids_hbm.at[pl.ds(2 * gr, 2)], idx_v)
        pltpu.sync_copy(tok_hbm.at[idx_v.at[0]], buf.at[pl.ds(0, CH)])
        pltpu.sync_copy(tok_hbm.at[idx_v.at[1]], buf.at[pl.ds(CH, CH)])
        pltpu.sync_copy(pos_hbm.at[pidx_v.at[0]], buf.at[pl.ds(0, CH)], add=True)
        pltpu.sync_copy(pos_hbm.at[pidx_v.at[1]], buf.at[pl.ds(CH, CH)], add=True)
        pltpu.sync_copy(buf, out_hbm.at[pl.ds(gr * S, S)])
        return carry

    lax.fori_loop(0, ROWS_PER_W, row, 0)


def kernel(input_ids, token_embedding, position_embedding):
    ids = input_ids.astype(jnp.int32).reshape(2 * B, CH)
    pos_ids = jnp.arange(S, dtype=jnp.int32).reshape(2, CH)
    mesh = plsc.VectorSubcoreMesh(core_axis_name="c", subcore_axis_name="s")
    out = pl.kernel(
        _body,
        out_type=jax.ShapeDtypeStruct((B * S, D), jnp.float32),
        mesh=mesh,
        scratch_types=[
            pltpu.VMEM((2, CH), jnp.int32),
            pltpu.VMEM((2, CH), jnp.int32),
            pltpu.VMEM((S, D), jnp.float32),
        ],
    )(ids, pos_ids, token_embedding, position_embedding)
    return out.reshape(B, S, D)


# speedup vs baseline: 2.3554x; 2.3554x over previous
import jax
import jax.numpy as jnp
from jax import lax
from jax.experimental import pallas as pl
from jax.experimental.pallas import tpu as pltpu, tpu_sc as plsc

D = 64
S = 200
B = 4096
NC = 2
NS = 16
NW = NC * NS
ROWS_PER_W = B // NW
CH = S // 2


def _body(ids_hbm, pos_ids_hbm, tok_hbm, pos_hbm, out_hbm, idx_v, pidx_v, buf):
    wid = lax.axis_index("s") * NC + lax.axis_index("c")
    row0 = wid * ROWS_PER_W
    pltpu.sync_copy(pos_ids_hbm, pidx_v)

    def row(r, carry):
        gr = row0 + r
        pltpu.sync_copy(ids_hbm.at[pl.ds(2 * gr, 2)], idx_v)
        pltpu.sync_copy(tok_hbm.at[idx_v.at[0]], buf.at[pl.ds(0, CH)])
        pltpu.sync_copy(tok_hbm.at[idx_v.at[1]], buf.at[pl.ds(CH, CH)])
        pltpu.sync_copy(pos_hbm.at[pidx_v.at[0]], buf.at[pl.ds(0, CH)], add=True)
        pltpu.sync_copy(pos_hbm.at[pidx_v.at[1]], buf.at[pl.ds(CH, CH)], add=True)
        pltpu.sync_copy(buf, out_hbm.at[pl.ds(gr * S, S)])
        return carry

    lax.fori_loop(0, ROWS_PER_W, row, 0)


def kernel(input_ids, token_embedding, position_embedding):
    ids = input_ids.astype(jnp.int32).reshape(2 * B, CH)
    pos_ids = jnp.arange(S, dtype=jnp.int32).reshape(2, CH)
    mesh = plsc.VectorSubcoreMesh(core_axis_name="c", subcore_axis_name="s")
    out = pl.kernel(
        _body,
        out_type=jax.ShapeDtypeStruct((B * S, D), jnp.float32),
        mesh=mesh,
        scratch_types=[
            pltpu.VMEM((2, CH), jnp.int32),
            pltpu.VMEM((2, CH), jnp.int32),
            pltpu.VMEM((S, D), jnp.float32),
        ],
        compiler_params=pltpu.CompilerParams(use_tc_tiling_on_sc=False),
    )(ids, pos_ids, token_embedding, position_embedding)
    return out.reshape(B, S, D)


# speedup vs baseline: 2.4760x; 1.0512x over previous
import jax
import jax.numpy as jnp
from jax import lax
from jax.experimental import pallas as pl
from jax.experimental.pallas import tpu as pltpu, tpu_sc as plsc

D = 64
S = 200
B = 4096
NC = 2
NS = 16
NW = NC * NS
ROWS = B // NW
CH = S // 2
NBUF = 3


def _body(ids_hbm, pos_ids_hbm, tok_hbm, pos_hbm, out_hbm,
          idx_all, pidx_v, buf, gsem, asem, wsem):
    wid = lax.axis_index("s") * NC + lax.axis_index("c")
    row0 = wid * ROWS
    pltpu.sync_copy(pos_ids_hbm, pidx_v)
    pltpu.sync_copy(ids_hbm.at[pl.ds(2 * row0, 2 * ROWS)], idx_all)

    def fire(t):
        s = lax.rem(t, NBUF)
        pltpu.async_copy(tok_hbm.at[idx_all.at[2 * t]],
                         buf.at[s, 0], gsem.at[s])
        pltpu.async_copy(tok_hbm.at[idx_all.at[2 * t + 1]],
                         buf.at[s, 1], gsem.at[s])

    def mid(t):
        s = lax.rem(t, NBUF)
        pltpu.make_async_copy(tok_hbm.at[idx_all.at[2 * t]],
                              buf.at[s, 0], gsem.at[s]).wait()
        pltpu.make_async_copy(tok_hbm.at[idx_all.at[2 * t + 1]],
                              buf.at[s, 1], gsem.at[s]).wait()
        pltpu.async_copy(pos_hbm.at[pidx_v.at[0]], buf.at[s, 0],
                         asem.at[s], add=True)
        pltpu.async_copy(pos_hbm.at[pidx_v.at[1]], buf.at[s, 1],
                         asem.at[s], add=True)

    def drain(t):
        s = lax.rem(t, NBUF)
        pltpu.make_async_copy(pos_hbm.at[pidx_v.at[0]], buf.at[s, 0],
                              asem.at[s]).wait()
        pltpu.make_async_copy(pos_hbm.at[pidx_v.at[1]], buf.at[s, 1],
                              asem.at[s]).wait()
        pltpu.async_copy(buf.at[s], out_hbm.at[pl.ds(2 * (row0 + t), 2)],
                         wsem.at[s])

    def flush(t):
        s = lax.rem(t, NBUF)
        pltpu.make_async_copy(buf.at[s], out_hbm.at[pl.ds(2 * (row0 + t), 2)],
                              wsem.at[s]).wait()

    def step(t, carry):
        pl.when(jnp.logical_and(t >= 2, t < ROWS + 2))(lambda: drain(t - 2))
        pl.when(t >= 3)(lambda: flush(t - 3))
        pl.when(t < ROWS)(lambda: fire(t))
        pl.when(jnp.logical_and(t >= 1, t < ROWS + 1))(lambda: mid(t - 1))
        return carry

    lax.fori_loop(0, ROWS + 3, step, 0)


def kernel(input_ids, token_embedding, position_embedding):
    ids = input_ids.astype(jnp.int32).reshape(2 * B, CH)
    pos_ids = jnp.arange(S, dtype=jnp.int32).reshape(2, CH)
    mesh = plsc.VectorSubcoreMesh(core_axis_name="c", subcore_axis_name="s")
    out = pl.kernel(
        _body,
        out_type=jax.ShapeDtypeStruct((2 * B, CH, D), jnp.float32),
        mesh=mesh,
        scratch_types=[
            pltpu.VMEM((2 * ROWS, CH), jnp.int32),
            pltpu.VMEM((2, CH), jnp.int32),
            pltpu.VMEM((NBUF, 2, CH, D), jnp.float32),
            pltpu.SemaphoreType.DMA((NBUF,)),
            pltpu.SemaphoreType.DMA((NBUF,)),
            pltpu.SemaphoreType.DMA((NBUF,)),
        ],
        compiler_params=pltpu.CompilerParams(use_tc_tiling_on_sc=False),
    )(ids, pos_ids, token_embedding, position_embedding)
    return out.reshape(B, S, D)
